# baseline (speedup 1.0000x reference)
import functools

import jax
import jax.numpy as jnp
from jax import lax
from jax.experimental import pallas as pl
from jax.experimental.pallas import tpu as pltpu
from jax.experimental.pallas import tpu_sc as plsc

N = 10000
E = 320000
F_IN = 128
HID = 384
NR = 256
NPAD = 10240

NUM_SC = 2
NUM_TILES = 16
LANES = 16
CPT = 4
CH = 8000

_mesh = functools.partial(
    plsc.VectorSubcoreMesh, core_axis_name="c", subcore_axis_name="s")


def _deg_body(edges_hbm, deg_hbm, ep_hbm, hist, srcbuf, dstbuf, packbuf, acc,
              tmp, shared):
    kind = lax.axis_index("c")
    sid = lax.axis_index("s")

    @pl.loop(0, NPAD // LANES)
    def _zero(i):
        hist[pl.ds(i * LANES, LANES)] = jnp.zeros((LANES,), jnp.float32)

    per_tile = E // NUM_TILES
    ones = jnp.ones((LANES,), jnp.float32)

    @pl.loop(0, per_tile // CH)
    def _chunk(ci):
        off = sid * per_tile + ci * CH
        pltpu.sync_copy(edges_hbm.at[pl.ds(off, CH)], srcbuf)
        pltpu.sync_copy(edges_hbm.at[pl.ds(E + off, CH)], dstbuf)

        @plsc.parallel_loop(0, CH // LANES, unroll=4)
        def _group(g):
            sl = pl.ds(g * LANES, LANES)
            sv = srcbuf[sl]
            dv = dstbuf[sl]
            idx = jnp.where(kind == 0, sv, dv)
            plsc.addupdate_scatter(hist, [idx], ones)
            packbuf[sl] = jnp.bitwise_or(sv, lax.shift_left(dv, 16))

        @pl.when(kind == 0)
        def _write_packed():
            pltpu.sync_copy(packbuf, ep_hbm.at[pl.ds(off, CH)])

    pltpu.sync_copy(hist, shared.at[sid])
    plsc.subcore_barrier()

    seg = NPAD // NUM_TILES
    base = sid * seg

    @pl.loop(0, seg // LANES)
    def _zacc(i):
        acc[pl.ds(i * LANES, LANES)] = jnp.zeros((LANES,), jnp.float32)

    for k in range(NUM_TILES):
        pltpu.sync_copy(shared.at[k, pl.ds(base, seg)], tmp)

        @pl.loop(0, seg // LANES)
        def _red(i):
            s = pl.ds(i * LANES, LANES)
            acc[s] = acc[s] + tmp[s]

    pltpu.sync_copy(acc, deg_hbm.at[pl.ds(kind * NPAD + base, seg)])


def _degrees(edge_index):
    kern = pl.kernel(
        _deg_body,
        out_type=(jax.ShapeDtypeStruct((2 * NPAD,), jnp.float32),
                  jax.ShapeDtypeStruct((E,), jnp.int32)),
        mesh=_mesh(),
        compiler_params=pltpu.CompilerParams(needs_layout_passes=False, use_tc_tiling_on_sc=False),
        scratch_types=[
            pltpu.VMEM((NPAD,), jnp.float32),
            pltpu.VMEM((CH,), jnp.int32),
            pltpu.VMEM((CH,), jnp.int32),
            pltpu.VMEM((CH,), jnp.int32),
            pltpu.VMEM((NPAD // NUM_TILES,), jnp.float32),
            pltpu.VMEM((NPAD // NUM_TILES,), jnp.float32),
            pltpu.VMEM_SHARED((NUM_TILES, NPAD), jnp.float32),
        ],
    )
    return kern(edge_index.reshape(2 * E))


def _seg_body(npasses, ep_hbm, xt_hbm, agg_hbm, xcols, aggcols, idxbuf, sem0,
              sem1):
    wid = lax.axis_index("s") * NUM_SC + lax.axis_index("c")

    for p in range(npasses):
        base = p * (NUM_SC * NUM_TILES * CPT) + wid * CPT
        pltpu.sync_copy(xt_hbm.at[pl.ds(base * NPAD, CPT * NPAD)], xcols)

        @plsc.parallel_loop(0, CPT * NPAD // LANES, unroll=8)
        def _zero(i):
            aggcols[pl.ds(i * LANES, LANES)] = jnp.zeros((LANES,), jnp.float32)

        def _process(buf_off):
            @plsc.parallel_loop(0, CH // LANES, unroll=8)
            def _group(g):
                w = idxbuf[pl.ds(buf_off + g * LANES, LANES)]
                src = jnp.bitwise_and(w, 0xFFFF)
                dst = lax.shift_right_logical(w, 16)
                for j in range(CPT):
                    v = plsc.load_gather(xcols, [src + j * NPAD])
                    plsc.addupdate_scatter(aggcols, [dst + j * NPAD], v)

        ncH = E // CH
        pltpu.async_copy(ep_hbm.at[pl.ds(0, CH)], idxbuf.at[pl.ds(0, CH)], sem0)

        @pl.loop(0, ncH, step=2)
        def _chunk(ci):
            pltpu.async_copy(
                ep_hbm.at[pl.ds((ci + 1) * CH, CH)],
                idxbuf.at[pl.ds(CH, CH)], sem1)
            pltpu.make_async_copy(
                ep_hbm.at[pl.ds(ci * CH, CH)],
                idxbuf.at[pl.ds(0, CH)], sem0).wait()
            _process(0)

            @pl.when(ci + 2 < ncH)
            def _prefetch():
                pltpu.async_copy(
                    ep_hbm.at[pl.ds((ci + 2) * CH, CH)],
                    idxbuf.at[pl.ds(0, CH)], sem0)

            pltpu.make_async_copy(
                ep_hbm.at[pl.ds((ci + 1) * CH, CH)],
                idxbuf.at[pl.ds(CH, CH)], sem1).wait()
            _process(CH)

        pltpu.sync_copy(aggcols, agg_hbm.at[pl.ds(base * NPAD, CPT * NPAD)])


def _segment_sum(epacked, xt, width):
    npasses = width // (NUM_SC * NUM_TILES * CPT)
    kern = pl.kernel(
        functools.partial(_seg_body, npasses),
        out_type=jax.ShapeDtypeStruct((width * NPAD,), jnp.float32),
        mesh=_mesh(),
        compiler_params=pltpu.CompilerParams(needs_layout_passes=False, use_tc_tiling_on_sc=False),
        scratch_types=[
            pltpu.VMEM((CPT * NPAD,), jnp.float32),
            pltpu.VMEM((CPT * NPAD,), jnp.float32),
            pltpu.VMEM((2 * CH,), jnp.int32),
            pltpu.SemaphoreType.DMA,
            pltpu.SemaphoreType.DMA,
        ],
    )
    return kern(epacked, xt.reshape(-1)).reshape(width, NPAD)


_TB = 256


def _prep_body(x_ref, deg_ref, ht_ref, s_ref):
    s = lax.rsqrt(jnp.maximum(deg_ref[...], 1.0))
    s_ref[...] = s
    r = lax.broadcasted_iota(jnp.int32, (_TB, _TB), 0)
    c = lax.broadcasted_iota(jnp.int32, (_TB, _TB), 1)
    eye = jnp.where(r == c, 1.0, 0.0).astype(jnp.float32)
    xt = lax.dot_general(x_ref[...], eye, (((0,), (0,)), ((), ())),
                         preferred_element_type=jnp.float32)
    ht_ref[...] = xt * s[0:1, :]


def _prepare(features_pad, deg):
    grid = NPAD // _TB
    return pl.pallas_call(
        _prep_body,
        grid=(grid,),
        in_specs=[
            pl.BlockSpec((_TB, F_IN), lambda j: (j, 0)),
            pl.BlockSpec((2, _TB), lambda j: (0, j)),
        ],
        out_specs=[
            pl.BlockSpec((F_IN, _TB), lambda j: (0, j)),
            pl.BlockSpec((2, _TB), lambda j: (0, j)),
        ],
        out_shape=[
            jax.ShapeDtypeStruct((F_IN, NPAD), jnp.float32),
            jax.ShapeDtypeStruct((2, NPAD), jnp.float32),
        ],
    )(features_pad, deg)


_MB = 512


def _mm_body(a_ref, s_ref, w1_ref, b1_ref, w2_ref, z_ref):
    s_in = s_ref[1:2, :]
    s_out = s_ref[0:1, :]
    a = a_ref[...] * s_in
    x1 = lax.dot_general(w1_ref[...], a, (((0,), (0,)), ((), ())),
                         preferred_element_type=jnp.float32)
    x1 = jnp.maximum(x1 + b1_ref[...], 0.0) * s_out
    z_ref[...] = lax.dot_general(w2_ref[...], x1, (((0,), (0,)), ((), ())),
                                 preferred_element_type=jnp.float32)


def _matmuls(agg1t, s, W1, b1, W2):
    grid = NPAD // _MB
    return pl.pallas_call(
        _mm_body,
        grid=(grid,),
        in_specs=[
            pl.BlockSpec((F_IN, _MB), lambda j: (0, j)),
            pl.BlockSpec((2, _MB), lambda j: (0, j)),
            pl.BlockSpec((F_IN, HID), lambda j: (0, 0)),
            pl.BlockSpec((HID, 1), lambda j: (0, 0)),
            pl.BlockSpec((HID, NR), lambda j: (0, 0)),
        ],
        out_specs=pl.BlockSpec((NR, _MB), lambda j: (0, j)),
        out_shape=jax.ShapeDtypeStruct((NR, NPAD), jnp.float32),
    )(agg1t, s, W1, b1.reshape(HID, 1), W2)


_EB = 128


def _epi_body(a_ref, s_ref, b2_ref, out_ref):
    a = a_ref[...] * s_ref[1:2, :]
    r = lax.broadcasted_iota(jnp.int32, (NR, NR), 0)
    c = lax.broadcasted_iota(jnp.int32, (NR, NR), 1)
    eye = jnp.where(r == c, 1.0, 0.0).astype(jnp.float32)
    out_ref[...] = lax.dot_general(a, eye, (((0,), (0,)), ((), ())),
                                   preferred_element_type=jnp.float32) + b2_ref[...]


def _epilogue(agg2t, s, b2):
    grid = (N + _EB - 1) // _EB
    return pl.pallas_call(
        _epi_body,
        grid=(grid,),
        in_specs=[
            pl.BlockSpec((NR, _EB), lambda j: (0, j)),
            pl.BlockSpec((2, _EB), lambda j: (0, j)),
            pl.BlockSpec((1, NR), lambda j: (0, 0)),
        ],
        out_specs=pl.BlockSpec((_EB, NR), lambda j: (j, 0)),
        out_shape=jax.ShapeDtypeStruct((N, NR), jnp.float32),
    )(agg2t, s, b2.reshape(1, NR))


def kernel(features, edge_index, W1, b1, W2, b2):
    deg, epacked = _degrees(edge_index.astype(jnp.int32))
    deg = deg.reshape(2, NPAD)
    ht, s = _prepare(features, deg)
    agg1t = _segment_sum(epacked, ht, F_IN)
    zt = _matmuls(agg1t, s, W1, b1, W2)
    agg2t = _segment_sum(epacked, zt, NR)
    out = _epilogue(agg2t, s, b2)
    return out.reshape(N, 16, 16)


# speedup vs baseline: 1.1794x; 1.1794x over previous
import functools

import jax
import jax.numpy as jnp
from jax import lax
from jax.experimental import pallas as pl
from jax.experimental.pallas import tpu as pltpu
from jax.experimental.pallas import tpu_sc as plsc

N = 10000
E = 320000
F_IN = 128
HID = 384
NR = 256
NPAD = 10240

NUM_SC = 2
NUM_TILES = 16
LANES = 16
CPT = 4
CH = 4000
CHS = 2000

_mesh = functools.partial(
    plsc.VectorSubcoreMesh, core_axis_name="c", subcore_axis_name="s")


def _deg_body(edges_hbm, deg_hbm, ep_hbm, hist, srcbuf, dstbuf, packbuf, acc,
              tmp, shared):
    kind = lax.axis_index("c")
    sid = lax.axis_index("s")

    @pl.loop(0, NPAD // LANES)
    def _zero(i):
        hist[pl.ds(i * LANES, LANES)] = jnp.zeros((LANES,), jnp.float32)

    per_tile = E // NUM_TILES
    ones = jnp.ones((LANES,), jnp.float32)

    @pl.loop(0, per_tile // CH)
    def _chunk(ci):
        off = sid * per_tile + ci * CH
        pltpu.sync_copy(edges_hbm.at[pl.ds(off, CH)], srcbuf)
        pltpu.sync_copy(edges_hbm.at[pl.ds(E + off, CH)], dstbuf)

        @plsc.parallel_loop(0, CH // LANES, unroll=4)
        def _group(g):
            sl = pl.ds(g * LANES, LANES)
            sv = srcbuf[sl]
            dv = dstbuf[sl]
            idx = jnp.where(kind == 0, sv, dv)
            plsc.addupdate_scatter(hist, [idx], ones)
            packbuf[sl] = jnp.bitwise_or(sv, lax.shift_left(dv, 16))

        @pl.when(kind == 0)
        def _write_packed():
            pltpu.sync_copy(packbuf, ep_hbm.at[pl.ds(off, CH)])

    pltpu.sync_copy(hist, shared.at[sid])
    plsc.subcore_barrier()

    seg = NPAD // NUM_TILES
    base = sid * seg

    @pl.loop(0, seg // LANES)
    def _zacc(i):
        acc[pl.ds(i * LANES, LANES)] = jnp.zeros((LANES,), jnp.float32)

    for k in range(NUM_TILES):
        pltpu.sync_copy(shared.at[k, pl.ds(base, seg)], tmp)

        @pl.loop(0, seg // LANES)
        def _red(i):
            s = pl.ds(i * LANES, LANES)
            acc[s] = acc[s] + tmp[s]

    pltpu.sync_copy(acc, deg_hbm.at[pl.ds(kind * NPAD + base, seg)])


def _degrees(edge_index):
    kern = pl.kernel(
        _deg_body,
        out_type=(jax.ShapeDtypeStruct((2 * NPAD,), jnp.float32),
                  jax.ShapeDtypeStruct((E,), jnp.int32)),
        mesh=_mesh(),
        compiler_params=pltpu.CompilerParams(needs_layout_passes=False, use_tc_tiling_on_sc=False),
        scratch_types=[
            pltpu.VMEM((NPAD,), jnp.float32),
            pltpu.VMEM((CH,), jnp.int32),
            pltpu.VMEM((CH,), jnp.int32),
            pltpu.VMEM((CH,), jnp.int32),
            pltpu.VMEM((NPAD // NUM_TILES,), jnp.float32),
            pltpu.VMEM((NPAD // NUM_TILES,), jnp.float32),
            pltpu.VMEM_SHARED((NUM_TILES, NPAD), jnp.float32),
        ],
    )
    return kern(edge_index.reshape(2 * E))


def _seg_body(pp, width, ep_hbm, xp_hbm, agg_hbm, xcols, aggcols, idxbuf, sem0,
              sem1):
    wid = lax.axis_index("s") * NUM_SC + lax.axis_index("c")
    base = wid * pp

    pltpu.sync_copy(xp_hbm.at[pl.ds(base * NPAD, pp * NPAD)], xcols)

    @plsc.parallel_loop(0, 2 * pp * NPAD // LANES, unroll=8)
    def _zero(i):
        aggcols[pl.ds(i * LANES, LANES)] = jnp.zeros((LANES,), jnp.float32)

    def _process(buf_off):
        @plsc.parallel_loop(0, CHS // LANES, unroll=4)
        def _group(g):
            w = idxbuf[pl.ds(buf_off + g * LANES, LANES)]
            src = jnp.bitwise_and(w, 0xFFFF)
            dst = lax.shift_right_logical(w, 16)
            for j in range(pp):
                wv = plsc.load_gather(xcols, [src + j * NPAD])
                bb = plsc.bitcast(wv, jnp.bfloat16)
                va, vb = plsc.unpack(bb, format=plsc.PackFormat.INTERLEAVED)
                plsc.addupdate_scatter(aggcols, [dst + j * NPAD], va)
                plsc.addupdate_scatter(aggcols, [dst + (pp + j) * NPAD], vb)

    ncH = E // CHS
    pltpu.async_copy(ep_hbm.at[pl.ds(0, CHS)], idxbuf.at[pl.ds(0, CHS)], sem0)

    @pl.loop(0, ncH, step=2)
    def _chunk(ci):
        pltpu.async_copy(
            ep_hbm.at[pl.ds((ci + 1) * CHS, CHS)],
            idxbuf.at[pl.ds(CHS, CHS)], sem1)
        pltpu.make_async_copy(
            ep_hbm.at[pl.ds(ci * CHS, CHS)],
            idxbuf.at[pl.ds(0, CHS)], sem0).wait()
        _process(0)

        @pl.when(ci + 2 < ncH)
        def _prefetch():
            pltpu.async_copy(
                ep_hbm.at[pl.ds((ci + 2) * CHS, CHS)],
                idxbuf.at[pl.ds(0, CHS)], sem0)

        pltpu.make_async_copy(
            ep_hbm.at[pl.ds((ci + 1) * CHS, CHS)],
            idxbuf.at[pl.ds(CHS, CHS)], sem1).wait()
        _process(CHS)

    pltpu.sync_copy(aggcols.at[pl.ds(0, pp * NPAD)],
                    agg_hbm.at[pl.ds(base * NPAD, pp * NPAD)])
    pltpu.sync_copy(aggcols.at[pl.ds(pp * NPAD, pp * NPAD)],
                    agg_hbm.at[pl.ds((width // 2 + base) * NPAD, pp * NPAD)])


def _segment_sum(epacked, xt_packed, width):
    pp = (width // 2) // (NUM_SC * NUM_TILES)
    kern = pl.kernel(
        functools.partial(_seg_body, pp, width),
        out_type=jax.ShapeDtypeStruct((width * NPAD,), jnp.float32),
        mesh=_mesh(),
        compiler_params=pltpu.CompilerParams(needs_layout_passes=False, use_tc_tiling_on_sc=False),
        scratch_types=[
            pltpu.VMEM((pp * NPAD,), jnp.int32),
            pltpu.VMEM((2 * pp * NPAD,), jnp.float32),
            pltpu.VMEM((2 * CHS,), jnp.int32),
            pltpu.SemaphoreType.DMA,
            pltpu.SemaphoreType.DMA,
        ],
    )
    return kern(epacked, xt_packed.reshape(-1)).reshape(width, NPAD)


_TB = 256


def _pack_rows(z):
    half = z.shape[0] // 2
    a = lax.bitcast_convert_type(z[:half].astype(jnp.bfloat16), jnp.uint16)
    b = lax.bitcast_convert_type(z[half:].astype(jnp.bfloat16), jnp.uint16)
    return (a.astype(jnp.int32)
            | lax.shift_left(b.astype(jnp.int32), 16))


def _prep_body(x_ref, deg_ref, ht_ref, s_ref):
    s = lax.rsqrt(jnp.maximum(deg_ref[...], 1.0))
    s_ref[...] = s
    r = lax.broadcasted_iota(jnp.int32, (_TB, _TB), 0)
    c = lax.broadcasted_iota(jnp.int32, (_TB, _TB), 1)
    eye = jnp.where(r == c, 1.0, 0.0).astype(jnp.float32)
    xt = lax.dot_general(x_ref[...], eye, (((0,), (0,)), ((), ())),
                         preferred_element_type=jnp.float32)
    ht_ref[...] = _pack_rows(xt * s[0:1, :])


def _prepare(features_pad, deg):
    grid = NPAD // _TB
    return pl.pallas_call(
        _prep_body,
        grid=(grid,),
        in_specs=[
            pl.BlockSpec((_TB, F_IN), lambda j: (j, 0)),
            pl.BlockSpec((2, _TB), lambda j: (0, j)),
        ],
        out_specs=[
            pl.BlockSpec((F_IN // 2, _TB), lambda j: (0, j)),
            pl.BlockSpec((2, _TB), lambda j: (0, j)),
        ],
        out_shape=[
            jax.ShapeDtypeStruct((F_IN // 2, NPAD), jnp.int32),
            jax.ShapeDtypeStruct((2, NPAD), jnp.float32),
        ],
    )(features_pad, deg)


_MB = 512


def _mm_body(a_ref, s_ref, w1_ref, b1_ref, w2_ref, z_ref):
    s_in = s_ref[1:2, :]
    s_out = s_ref[0:1, :]
    a = a_ref[...] * s_in
    x1 = lax.dot_general(w1_ref[...], a, (((0,), (0,)), ((), ())),
                         preferred_element_type=jnp.float32)
    x1 = jnp.maximum(x1 + b1_ref[...], 0.0) * s_out
    z = lax.dot_general(w2_ref[...], x1, (((0,), (0,)), ((), ())),
                        preferred_element_type=jnp.float32)
    z_ref[...] = _pack_rows(z)


def _matmuls(agg1t, s, W1, b1, W2):
    grid = NPAD // _MB
    return pl.pallas_call(
        _mm_body,
        grid=(grid,),
        in_specs=[
            pl.BlockSpec((F_IN, _MB), lambda j: (0, j)),
            pl.BlockSpec((2, _MB), lambda j: (0, j)),
            pl.BlockSpec((F_IN, HID), lambda j: (0, 0)),
            pl.BlockSpec((HID, 1), lambda j: (0, 0)),
            pl.BlockSpec((HID, NR), lambda j: (0, 0)),
        ],
        out_specs=pl.BlockSpec((NR // 2, _MB), lambda j: (0, j)),
        out_shape=jax.ShapeDtypeStruct((NR // 2, NPAD), jnp.int32),
    )(agg1t, s, W1, b1.reshape(HID, 1), W2)


_EB = 128


def _epi_body(a_ref, s_ref, b2_ref, out_ref):
    a = a_ref[...] * s_ref[1:2, :]
    r = lax.broadcasted_iota(jnp.int32, (NR, NR), 0)
    c = lax.broadcasted_iota(jnp.int32, (NR, NR), 1)
    eye = jnp.where(r == c, 1.0, 0.0).astype(jnp.float32)
    out_ref[...] = lax.dot_general(a, eye, (((0,), (0,)), ((), ())),
                                   preferred_element_type=jnp.float32) + b2_ref[...]


def _epilogue(agg2t, s, b2):
    grid = (N + _EB - 1) // _EB
    return pl.pallas_call(
        _epi_body,
        grid=(grid,),
        in_specs=[
            pl.BlockSpec((NR, _EB), lambda j: (0, j)),
            pl.BlockSpec((2, _EB), lambda j: (0, j)),
            pl.BlockSpec((1, NR), lambda j: (0, 0)),
        ],
        out_specs=pl.BlockSpec((_EB, NR), lambda j: (j, 0)),
        out_shape=jax.ShapeDtypeStruct((N, NR), jnp.float32),
    )(agg2t, s, b2.reshape(1, NR))


def kernel(features, edge_index, W1, b1, W2, b2):
    deg, epacked = _degrees(edge_index.astype(jnp.int32))
    deg = deg.reshape(2, NPAD)
    ht, s = _prepare(features, deg)
    agg1t = _segment_sum(epacked, ht, F_IN)
    zt = _matmuls(agg1t, s, W1, b1, W2)
    agg2t = _segment_sum(epacked, zt, NR)
    out = _epilogue(agg2t, s, b2)
    return out.reshape(N, 16, 16)
